# baseline (speedup 1.0000x reference)
import jax
import jax.numpy as jnp
from jax.experimental import pallas as pl
from jax.experimental.pallas import tpu as pltpu

_CHUNK_ROWS = 1024
_SLOTS = 6


def _pipelined_copy(x_hbm, o_hbm, buf, in_sems, out_sems):
    n = x_hbm.shape[0] // _CHUNK_ROWS

    def in_copy(i):
        return pltpu.make_async_copy(
            x_hbm.at[pl.ds(i * _CHUNK_ROWS, _CHUNK_ROWS), :],
            buf.at[i % _SLOTS],
            in_sems.at[i % _SLOTS],
        )

    def out_copy(i):
        return pltpu.make_async_copy(
            buf.at[i % _SLOTS],
            o_hbm.at[pl.ds(i * _CHUNK_ROWS, _CHUNK_ROWS), :],
            out_sems.at[i % _SLOTS],
        )

    for i in range(min(_SLOTS, n)):
        in_copy(i).start()
    for i in range(n):
        in_copy(i).wait()
        out_copy(i).start()
        j = i - (_SLOTS - 1)
        if 0 <= j and j + _SLOTS < n:
            out_copy(j).wait()
            in_copy(j + _SLOTS).start()
    for j in range(max(0, n - _SLOTS), n):
        out_copy(j).wait()


def kernel(schedule_context):
    rows, cols = schedule_context.shape
    return pl.pallas_call(
        _pipelined_copy,
        in_specs=[pl.BlockSpec(memory_space=pl.ANY)],
        out_specs=pl.BlockSpec(memory_space=pl.ANY),
        out_shape=jax.ShapeDtypeStruct((rows, cols), schedule_context.dtype),
        scratch_shapes=[
            pltpu.VMEM((_SLOTS, _CHUNK_ROWS, cols), schedule_context.dtype),
            pltpu.SemaphoreType.DMA((_SLOTS,)),
            pltpu.SemaphoreType.DMA((_SLOTS,)),
        ],
    )(schedule_context)


# speedup vs baseline: 1.1985x; 1.1985x over previous
import jax
import jax.numpy as jnp
from jax.experimental import pallas as pl
from jax.experimental.pallas import tpu as pltpu


def _copy_block(x_ref, o_ref):
    o_ref[...] = x_ref[...]


def kernel(schedule_context):
    rows, cols = schedule_context.shape
    block_rows = 1536
    return pl.pallas_call(
        _copy_block,
        grid=(pl.cdiv(rows, block_rows),),
        in_specs=[pl.BlockSpec((block_rows, cols), lambda i: (i, 0))],
        out_specs=pl.BlockSpec((block_rows, cols), lambda i: (i, 0)),
        out_shape=jax.ShapeDtypeStruct((rows, cols), schedule_context.dtype),
        compiler_params=pltpu.CompilerParams(
            dimension_semantics=("parallel",),
            vmem_limit_bytes=128 * 1024 * 1024,
        ),
    )(schedule_context)


# speedup vs baseline: 1.2002x; 1.0014x over previous
import jax
import jax.numpy as jnp
from jax.experimental import pallas as pl
from jax.experimental.pallas import tpu as pltpu


def _copy_block(x_ref, o_ref):
    o_ref[...] = x_ref[...]


def kernel(schedule_context):
    rows, cols = schedule_context.shape
    block_rows = 1792
    return pl.pallas_call(
        _copy_block,
        grid=(pl.cdiv(rows, block_rows),),
        in_specs=[pl.BlockSpec((block_rows, cols), lambda i: (i, 0))],
        out_specs=pl.BlockSpec((block_rows, cols), lambda i: (i, 0)),
        out_shape=jax.ShapeDtypeStruct((rows, cols), schedule_context.dtype),
        compiler_params=pltpu.CompilerParams(
            dimension_semantics=("parallel",),
            vmem_limit_bytes=128 * 1024 * 1024,
        ),
    )(schedule_context)


# speedup vs baseline: 1.2006x; 1.0004x over previous
import jax
import jax.numpy as jnp
from jax.experimental import pallas as pl
from jax.experimental.pallas import tpu as pltpu


def _copy_block(x_ref, o_ref):
    o_ref[...] = x_ref[...]


def kernel(schedule_context):
    rows, cols = schedule_context.shape
    block_rows = 1984
    return pl.pallas_call(
        _copy_block,
        grid=(pl.cdiv(rows, block_rows),),
        in_specs=[pl.BlockSpec((block_rows, cols), lambda i: (i, 0))],
        out_specs=pl.BlockSpec((block_rows, cols), lambda i: (i, 0)),
        out_shape=jax.ShapeDtypeStruct((rows, cols), schedule_context.dtype),
        compiler_params=pltpu.CompilerParams(
            dimension_semantics=("parallel",),
            vmem_limit_bytes=128 * 1024 * 1024,
        ),
    )(schedule_context)


# speedup vs baseline: 1.2014x; 1.0006x over previous
import jax
import jax.numpy as jnp
from jax.experimental import pallas as pl
from jax.experimental.pallas import tpu as pltpu


def _copy_block(x_ref, o_ref):
    o_ref[...] = x_ref[...]


def kernel(schedule_context):
    rows, cols = schedule_context.shape
    block_rows = 2016
    return pl.pallas_call(
        _copy_block,
        grid=(pl.cdiv(rows, block_rows),),
        in_specs=[pl.BlockSpec((block_rows, cols), lambda i: (i, 0))],
        out_specs=pl.BlockSpec((block_rows, cols), lambda i: (i, 0)),
        out_shape=jax.ShapeDtypeStruct((rows, cols), schedule_context.dtype),
        compiler_params=pltpu.CompilerParams(
            dimension_semantics=("parallel",),
            vmem_limit_bytes=128 * 1024 * 1024,
        ),
    )(schedule_context)
